# baseline (device time: 69202 ns/iter reference)
import jax
import jax.numpy as jnp
from jax import lax
from jax.experimental import pallas as pl
from jax.experimental.pallas import tpu as pltpu

N_DEV = 4
SCALE = 0.08838834764831843


def kernel(x, Wq, Wo, K_ext, V_ext):
    B, Sq, D = x.shape
    _, Skv, Hl, Dh = K_ext.shape
    Dq = Wq.shape[1]
    BLK = Sq // N_DEV

    KT = K_ext[0].transpose(1, 2, 0)
    VT = V_ext[0].transpose(1, 0, 2)

    def body(x_ref, wq_ref, wo_ref, kt_ref, vt_ref, out_ref,
             q_ref, attn_ref, p_ref,
             sbuf_ref, rbuf_ref, fin_ref, agl_ref, agr_ref, agd_ref,
             rs_send, rs_recv, ag_send, ag_recv):
        my = lax.axis_index("i")
        left = lax.rem(my + N_DEV - 1, N_DEV)
        right = lax.rem(my + 1, N_DEV)

        barrier = pltpu.get_barrier_semaphore()
        for nbr in (left, right):
            pl.semaphore_signal(barrier, inc=1, device_id=(nbr,),
                                device_id_type=pl.DeviceIdType.MESH)
        pl.semaphore_wait(barrier, 2)

        q_ref[...] = jnp.dot(x_ref[0], wq_ref[...],
                             preferred_element_type=jnp.float32) * SCALE

        def compute_block(rel):
            babs = lax.rem(my + rel, N_DEV)
            qblk = q_ref[pl.ds(babs * BLK, BLK), :]
            for h in range(Hl):
                qh = qblk[:, h * Dh:(h + 1) * Dh]
                s = jnp.dot(qh, kt_ref[h],
                            preferred_element_type=jnp.float32)
                p = jnp.exp(s)
                l = jnp.sum(p, axis=1, keepdims=True)
                oh = jnp.dot(p, vt_ref[h],
                             preferred_element_type=jnp.float32)
                attn_ref[:, h * Dh:(h + 1) * Dh] = oh / l
            p_ref[rel] = jnp.dot(attn_ref[...], wo_ref[...],
                                 preferred_element_type=jnp.float32)

        def rs_rdma(step, src):
            return pltpu.make_async_remote_copy(
                src_ref=src, dst_ref=rbuf_ref.at[step],
                send_sem=rs_send.at[step], recv_sem=rs_recv.at[step],
                device_id=(right,), device_id_type=pl.DeviceIdType.MESH)

        compute_block(3)
        rs0 = rs_rdma(0, p_ref.at[3])
        rs0.start()
        compute_block(2)
        rs0.wait()
        sbuf_ref[0] = rbuf_ref[0] + p_ref[2]
        rs1 = rs_rdma(1, sbuf_ref.at[0])
        rs1.start()
        compute_block(1)
        rs1.wait()
        sbuf_ref[1] = rbuf_ref[1] + p_ref[1]
        rs2 = rs_rdma(2, sbuf_ref.at[1])
        rs2.start()
        compute_block(0)
        rs2.wait()
        fin_ref[...] = rbuf_ref[2] + p_ref[0]
        out_ref[0, pl.ds(my * BLK, BLK), :] = fin_ref[...]

        ag_r1 = pltpu.make_async_remote_copy(
            src_ref=fin_ref, dst_ref=agl_ref,
            send_sem=ag_send.at[0], recv_sem=ag_recv.at[0],
            device_id=(right,), device_id_type=pl.DeviceIdType.MESH)
        ag_l1 = pltpu.make_async_remote_copy(
            src_ref=fin_ref, dst_ref=agr_ref,
            send_sem=ag_send.at[1], recv_sem=ag_recv.at[1],
            device_id=(left,), device_id_type=pl.DeviceIdType.MESH)
        ag_r1.start()
        ag_l1.start()
        ag_r1.wait()
        ag_l1.wait()
        out_ref[0, pl.ds(left * BLK, BLK), :] = agl_ref[...]
        out_ref[0, pl.ds(right * BLK, BLK), :] = agr_ref[...]
        half = D // 2
        ag2r = pltpu.make_async_remote_copy(
            src_ref=agl_ref.at[:, :half], dst_ref=agd_ref.at[:, :half],
            send_sem=ag_send.at[2], recv_sem=ag_recv.at[2],
            device_id=(right,), device_id_type=pl.DeviceIdType.MESH)
        ag2l = pltpu.make_async_remote_copy(
            src_ref=agr_ref.at[:, half:], dst_ref=agd_ref.at[:, half:],
            send_sem=ag_send.at[3], recv_sem=ag_recv.at[3],
            device_id=(left,), device_id_type=pl.DeviceIdType.MESH)
        ag2r.start()
        ag2l.start()
        ag2r.wait()
        ag2l.wait()
        out_ref[0, pl.ds(lax.rem(my + 2, N_DEV) * BLK, BLK), :] = agd_ref[...]

    return pl.pallas_call(
        body,
        out_shape=jax.ShapeDtypeStruct((B, Sq, D), jnp.float32),
        in_specs=[
            pl.BlockSpec(memory_space=pltpu.VMEM),
            pl.BlockSpec(memory_space=pltpu.VMEM),
            pl.BlockSpec(memory_space=pltpu.VMEM),
            pl.BlockSpec(memory_space=pltpu.VMEM),
            pl.BlockSpec(memory_space=pltpu.VMEM),
        ],
        out_specs=pl.BlockSpec(memory_space=pltpu.VMEM),
        scratch_shapes=[
            pltpu.VMEM((Sq, Dq), jnp.float32),
            pltpu.VMEM((BLK, Dq), jnp.float32),
            pltpu.VMEM((N_DEV, BLK, D), jnp.float32),
            pltpu.VMEM((2, BLK, D), jnp.float32),
            pltpu.VMEM((3, BLK, D), jnp.float32),
            pltpu.VMEM((BLK, D), jnp.float32),
            pltpu.VMEM((BLK, D), jnp.float32),
            pltpu.VMEM((BLK, D), jnp.float32),
            pltpu.VMEM((BLK, D), jnp.float32),
            pltpu.SemaphoreType.DMA((3,)),
            pltpu.SemaphoreType.DMA((3,)),
            pltpu.SemaphoreType.DMA((4,)),
            pltpu.SemaphoreType.DMA((4,)),
        ],
        compiler_params=pltpu.CompilerParams(
            collective_id=0,
            vmem_limit_bytes=60 * 1024 * 1024,
        ),
    )(x, Wq, Wo, KT, VT)


# device time: 53561 ns/iter; 1.2920x vs baseline; 1.2920x over previous
import jax
import jax.numpy as jnp
from jax import lax
from jax.experimental import pallas as pl
from jax.experimental.pallas import tpu as pltpu

N_DEV = 4
SCALE = 0.08838834764831843


def kernel(x, Wq, Wo, K_ext, V_ext):
    B, Sq, D = x.shape
    _, Skv, Hl, Dh = K_ext.shape
    Dq = Wq.shape[1]
    BLK = Sq // N_DEV

    def body(x_ref, wq_ref, wo_ref, k_ref, v_ref, out_ref,
             attn_ref, p_ref, kvm_ref, vvm_ref,
             sbuf_ref, rbuf_ref, fin_ref, agl_ref, agr_ref, agd_ref,
             stage_sems, rs_send, rs_recv, ag_send, ag_recv):
        my = lax.axis_index("i")
        left = lax.rem(my + N_DEV - 1, N_DEV)
        right = lax.rem(my + 1, N_DEV)

        barrier = pltpu.get_barrier_semaphore()
        for nbr in (left, right):
            pl.semaphore_signal(barrier, inc=1, device_id=(nbr,),
                                device_id_type=pl.DeviceIdType.MESH)
        pl.semaphore_wait(barrier, 2)

        stage = []
        for h in range(Hl):
            kc = pltpu.make_async_copy(
                k_ref.at[0, :, h, :], kvm_ref.at[h], stage_sems.at[h])
            vc = pltpu.make_async_copy(
                v_ref.at[0, :, h, :], vvm_ref.at[h], stage_sems.at[Hl + h])
            kc.start()
            vc.start()
            stage.append((kc, vc))

        def compute_block(rel, first):
            babs = lax.rem(my + rel, N_DEV)
            qblk = jnp.dot(x_ref[0, pl.ds(babs * BLK, BLK), :], wq_ref[...],
                           preferred_element_type=jnp.float32) * SCALE
            for h in range(Hl):
                if first:
                    kc, vc = stage[h]
                    kc.wait()
                    vc.wait()
                qh = qblk[:, h * Dh:(h + 1) * Dh]
                s = lax.dot_general(
                    qh, kvm_ref[h], (((1,), (1,)), ((), ())),
                    preferred_element_type=jnp.float32)
                p = jnp.exp(s)
                l = jnp.sum(p, axis=1, keepdims=True)
                oh = jnp.dot(p, vvm_ref[h],
                             preferred_element_type=jnp.float32)
                attn_ref[:, h * Dh:(h + 1) * Dh] = oh / l
            p_ref[rel] = jnp.dot(attn_ref[...], wo_ref[...],
                                 preferred_element_type=jnp.float32)

        def rs_rdma(step, src):
            return pltpu.make_async_remote_copy(
                src_ref=src, dst_ref=rbuf_ref.at[step],
                send_sem=rs_send.at[step], recv_sem=rs_recv.at[step],
                device_id=(right,), device_id_type=pl.DeviceIdType.MESH)

        compute_block(3, True)
        rs0 = rs_rdma(0, p_ref.at[3])
        rs0.start()
        compute_block(2, False)
        rs0.wait()
        sbuf_ref[0] = rbuf_ref[0] + p_ref[2]
        rs1 = rs_rdma(1, sbuf_ref.at[0])
        rs1.start()
        compute_block(1, False)
        rs1.wait()
        sbuf_ref[1] = rbuf_ref[1] + p_ref[1]
        rs2 = rs_rdma(2, sbuf_ref.at[1])
        rs2.start()
        compute_block(0, False)
        rs2.wait()
        fin_ref[...] = rbuf_ref[2] + p_ref[0]
        out_ref[0, pl.ds(my * BLK, BLK), :] = fin_ref[...]

        ag_r1 = pltpu.make_async_remote_copy(
            src_ref=fin_ref, dst_ref=agl_ref,
            send_sem=ag_send.at[0], recv_sem=ag_recv.at[0],
            device_id=(right,), device_id_type=pl.DeviceIdType.MESH)
        ag_l1 = pltpu.make_async_remote_copy(
            src_ref=fin_ref, dst_ref=agr_ref,
            send_sem=ag_send.at[1], recv_sem=ag_recv.at[1],
            device_id=(left,), device_id_type=pl.DeviceIdType.MESH)
        ag_r1.start()
        ag_l1.start()
        ag_r1.wait()
        ag_l1.wait()
        out_ref[0, pl.ds(left * BLK, BLK), :] = agl_ref[...]
        out_ref[0, pl.ds(right * BLK, BLK), :] = agr_ref[...]
        half = D // 2
        ag2r = pltpu.make_async_remote_copy(
            src_ref=agl_ref.at[:, :half], dst_ref=agd_ref.at[:, :half],
            send_sem=ag_send.at[2], recv_sem=ag_recv.at[2],
            device_id=(right,), device_id_type=pl.DeviceIdType.MESH)
        ag2l = pltpu.make_async_remote_copy(
            src_ref=agr_ref.at[:, half:], dst_ref=agd_ref.at[:, half:],
            send_sem=ag_send.at[3], recv_sem=ag_recv.at[3],
            device_id=(left,), device_id_type=pl.DeviceIdType.MESH)
        ag2r.start()
        ag2l.start()
        ag2r.wait()
        ag2l.wait()
        out_ref[0, pl.ds(lax.rem(my + 2, N_DEV) * BLK, BLK), :] = agd_ref[...]

    return pl.pallas_call(
        body,
        out_shape=jax.ShapeDtypeStruct((B, Sq, D), jnp.float32),
        in_specs=[
            pl.BlockSpec(memory_space=pltpu.VMEM),
            pl.BlockSpec(memory_space=pltpu.VMEM),
            pl.BlockSpec(memory_space=pltpu.VMEM),
            pl.BlockSpec(memory_space=pl.ANY),
            pl.BlockSpec(memory_space=pl.ANY),
        ],
        out_specs=pl.BlockSpec(memory_space=pltpu.VMEM),
        scratch_shapes=[
            pltpu.VMEM((BLK, Dq), jnp.float32),
            pltpu.VMEM((N_DEV, BLK, D), jnp.float32),
            pltpu.VMEM((Hl, Skv, Dh), jnp.float32),
            pltpu.VMEM((Hl, Skv, Dh), jnp.float32),
            pltpu.VMEM((2, BLK, D), jnp.float32),
            pltpu.VMEM((3, BLK, D), jnp.float32),
            pltpu.VMEM((BLK, D), jnp.float32),
            pltpu.VMEM((BLK, D), jnp.float32),
            pltpu.VMEM((BLK, D), jnp.float32),
            pltpu.VMEM((BLK, D), jnp.float32),
            pltpu.SemaphoreType.DMA((2 * Hl,)),
            pltpu.SemaphoreType.DMA((3,)),
            pltpu.SemaphoreType.DMA((3,)),
            pltpu.SemaphoreType.DMA((4,)),
            pltpu.SemaphoreType.DMA((4,)),
        ],
        compiler_params=pltpu.CompilerParams(
            collective_id=0,
            vmem_limit_bytes=60 * 1024 * 1024,
        ),
    )(x, Wq, Wo, K_ext, V_ext)
